# baseline (speedup 1.0000x reference)
import functools

import jax
import jax.numpy as jnp
from jax import lax
from jax.experimental import pallas as pl
from jax.experimental.pallas import tpu as pltpu
from jax.experimental.pallas import tpu_sc as plsc

VOCAB = 100000
EMBED_DIM = 64
BATCH = 4096
HIST = 200

NUM_CORES = 2
NUM_SUBCORES = 16
LANES = 16
NUM_WORKERS = NUM_CORES * NUM_SUBCORES
ROWS_PER_W = BATCH // NUM_WORKERS
ROWS_PER_CHUNK = 64
IDX_PER_W = ROWS_PER_W * HIST
GROUPS = ROWS_PER_W // LANES

_S_BLOCK = 8192
HALF_WORDS = 65536
_S_GRID = HALF_WORDS // _S_BLOCK
_EMB_BLOCKS = (VOCAB + _S_BLOCK - 1) // _S_BLOCK


def _bf16_bits(s):
    bits = jax.lax.bitcast_convert_type(s, jnp.int32)
    rne = jnp.int32(0x7FFF) + (jax.lax.shift_right_logical(bits, 16) & 1)
    return jax.lax.shift_right_logical(bits + rne, 16)


def _s_body(eA_ref, eB_ref, w_ref, b_ref, out_ref):
    w = w_ref[...]

    def s_of(e_ref):
        s = jax.lax.dot_general(
            w, e_ref[...], (((1,), (0,)), ((), ())),
            preferred_element_type=jnp.float32)
        return ((s + b_ref[0]) * (1.0 / HIST)).reshape(_S_BLOCK)

    lo = _bf16_bits(s_of(eA_ref))
    hi = _bf16_bits(s_of(eB_ref))
    out_ref[...] = lo | jax.lax.shift_left(hi, 16)


def _compute_s(emb, W, b):
    return pl.pallas_call(
        _s_body,
        grid=(_S_GRID,),
        in_specs=[
            pl.BlockSpec((EMBED_DIM, _S_BLOCK), lambda i: (0, i)),
            pl.BlockSpec(
                (EMBED_DIM, _S_BLOCK),
                lambda i: (0, jnp.minimum(i + _S_GRID, _EMB_BLOCKS - 1))),
            pl.BlockSpec((1, EMBED_DIM), lambda i: (0, 0)),
            pl.BlockSpec(memory_space=pltpu.SMEM),
        ],
        out_specs=pl.BlockSpec((_S_BLOCK,), lambda i: (i,)),
        out_shape=jax.ShapeDtypeStruct((HALF_WORDS,), jnp.int32),
    )(emb.T, emb.T, W, b)


def _make_sc_kernel(interpret=False):
    mesh = plsc.VectorSubcoreMesh(
        core_axis_name="c", subcore_axis_name="s",
        num_cores=NUM_CORES, num_subcores=NUM_SUBCORES)

    @functools.partial(
        pl.kernel,
        mesh=mesh,
        out_type=jax.ShapeDtypeStruct((BATCH,), jnp.float32),
        scratch_types=[
            pltpu.VMEM((HALF_WORDS,), jnp.int32),
            pltpu.VMEM((HIST, ROWS_PER_W), jnp.int32),
            pltpu.VMEM((ROWS_PER_W,), jnp.float32),
            pltpu.SemaphoreType.DMA,
            pltpu.SemaphoreType.DMA,
        ],
        compiler_params=pltpu.CompilerParams(needs_layout_passes=False),
        interpret=interpret,
    )
    def sc_kernel(s_hbm, xT_hbm, out_hbm, table_v, idx_v, out_v,
                  sem_t, sem_i):
        wid = lax.axis_index("s") * NUM_CORES + lax.axis_index("c")
        cp_t = pltpu.async_copy(s_hbm, table_v, sem_t)
        cp_i = pltpu.async_copy(
            xT_hbm.at[:, pl.ds(wid * ROWS_PER_W, ROWS_PER_W)], idx_v, sem_i)
        cp_i.wait()
        cp_t.wait()
        def group(g, _):
            def body(l, acc):
                ind = idx_v[l, pl.ds(g * LANES, LANES)]
                word = plsc.load_gather(table_v, [ind & 0xFFFF])
                sh = lax.shift_left(lax.shift_right_logical(ind, 16), 4)
                f32b = lax.shift_left(lax.shift_right_logical(word, sh), 16)
                return acc + plsc.bitcast(f32b, jnp.float32)

            acc = lax.fori_loop(0, HIST, body,
                                jnp.zeros((LANES,), jnp.float32),
                                unroll=16)
            out_v[pl.ds(g * LANES, LANES)] = 1.0 / (1.0 + jnp.exp(-acc))
            return 0

        lax.fori_loop(0, GROUPS, group, 0)
        pltpu.sync_copy(
            out_v, out_hbm.at[pl.ds(wid * ROWS_PER_W, ROWS_PER_W)])

    return sc_kernel


_sc_kernel_cache = {}


def _get_sc_kernel():
    if "k" not in _sc_kernel_cache:
        _sc_kernel_cache["k"] = _make_sc_kernel()
    return _sc_kernel_cache["k"]


def kernel(x, emb, W, b):
    s = _compute_s(emb, W, b)
    out = _get_sc_kernel()(s, x.T)
    return out.reshape(BATCH, 1)


# speedup vs baseline: 1.1416x; 1.1416x over previous
import functools

import jax
import jax.numpy as jnp
from jax import lax
from jax.experimental import pallas as pl
from jax.experimental.pallas import tpu as pltpu
from jax.experimental.pallas import tpu_sc as plsc

VOCAB = 100000
EMBED_DIM = 64
BATCH = 4096
HIST = 200

NUM_CORES = 2
NUM_SUBCORES = 16
LANES = 16
NUM_WORKERS = NUM_CORES * NUM_SUBCORES
ROWS_PER_W = BATCH // NUM_WORKERS
ROWS_PER_CHUNK = 64
IDX_PER_W = ROWS_PER_W * HIST
GROUPS = ROWS_PER_W // LANES

_S_BLOCK = 8192
HALF_WORDS = 65536
_S_GRID = HALF_WORDS // _S_BLOCK
_EMB_BLOCKS = (VOCAB + _S_BLOCK - 1) // _S_BLOCK


def _bf16_bits(s):
    bits = jax.lax.bitcast_convert_type(s, jnp.int32)
    rne = jnp.int32(0x7FFF) + (jax.lax.shift_right_logical(bits, 16) & 1)
    return jax.lax.shift_right_logical(bits + rne, 16)


def _s_body(eA_ref, eB_ref, w_ref, b_ref, out_ref):
    w = w_ref[...]

    def s_of(e_ref):
        s = jax.lax.dot_general(
            w, e_ref[...], (((1,), (0,)), ((), ())),
            preferred_element_type=jnp.float32)
        return ((s + b_ref[0]) * (1.0 / HIST)).reshape(_S_BLOCK)

    lo = _bf16_bits(s_of(eA_ref))
    hi = _bf16_bits(s_of(eB_ref))
    out_ref[...] = lo | jax.lax.shift_left(hi, 16)


def _compute_s(emb, W, b):
    return pl.pallas_call(
        _s_body,
        grid=(_S_GRID,),
        in_specs=[
            pl.BlockSpec((EMBED_DIM, _S_BLOCK), lambda i: (0, i)),
            pl.BlockSpec(
                (EMBED_DIM, _S_BLOCK),
                lambda i: (0, jnp.minimum(i + _S_GRID, _EMB_BLOCKS - 1))),
            pl.BlockSpec((1, EMBED_DIM), lambda i: (0, 0)),
            pl.BlockSpec(memory_space=pltpu.SMEM),
        ],
        out_specs=pl.BlockSpec((_S_BLOCK,), lambda i: (i,)),
        out_shape=jax.ShapeDtypeStruct((HALF_WORDS,), jnp.int32),
    )(emb.T, emb.T, W, b)


def _make_sc_kernel(interpret=False):
    mesh = plsc.VectorSubcoreMesh(
        core_axis_name="c", subcore_axis_name="s",
        num_cores=NUM_CORES, num_subcores=NUM_SUBCORES)

    @functools.partial(
        pl.kernel,
        mesh=mesh,
        out_type=jax.ShapeDtypeStruct((BATCH,), jnp.float32),
        scratch_types=[
            pltpu.VMEM((HALF_WORDS,), jnp.int32),
            pltpu.VMEM((HIST, ROWS_PER_W), jnp.int32),
            pltpu.VMEM((ROWS_PER_W,), jnp.float32),
            pltpu.VMEM_SHARED((HALF_WORDS,), jnp.int32),
            pltpu.SemaphoreType.DMA,
            pltpu.SemaphoreType.DMA,
        ],
        compiler_params=pltpu.CompilerParams(needs_layout_passes=False),
        interpret=interpret,
    )
    def sc_kernel(s_hbm, xT_hbm, out_hbm, table_v, idx_v, out_v,
                  table_sh, sem_t, sem_i):
        sid = lax.axis_index("s")
        wid = sid * NUM_CORES + lax.axis_index("c")
        cp_i = pltpu.async_copy(
            xT_hbm.at[:, pl.ds(wid * ROWS_PER_W, ROWS_PER_W)], idx_v, sem_i)
        @pl.when(sid == 0)
        def _():
            pltpu.sync_copy(s_hbm, table_sh)
        plsc.subcore_barrier()
        cp_t = pltpu.async_copy(table_sh, table_v, sem_t)
        cp_i.wait()
        cp_t.wait()
        def group(g, _):
            def body(l, acc):
                ind = idx_v[l, pl.ds(g * LANES, LANES)]
                word = plsc.load_gather(table_v, [ind & 0xFFFF])
                sh = lax.shift_left(lax.shift_right_logical(ind, 16), 4)
                f32b = lax.shift_left(lax.shift_right_logical(word, sh), 16)
                return acc + plsc.bitcast(f32b, jnp.float32)

            acc = lax.fori_loop(0, HIST, body,
                                jnp.zeros((LANES,), jnp.float32),
                                unroll=8)
            out_v[pl.ds(g * LANES, LANES)] = 1.0 / (1.0 + jnp.exp(-acc))
            return 0

        lax.fori_loop(0, GROUPS, group, 0)
        pltpu.sync_copy(
            out_v, out_hbm.at[pl.ds(wid * ROWS_PER_W, ROWS_PER_W)])

    return sc_kernel


_sc_kernel_cache = {}


def _get_sc_kernel():
    if "k" not in _sc_kernel_cache:
        _sc_kernel_cache["k"] = _make_sc_kernel()
    return _sc_kernel_cache["k"]


def kernel(x, emb, W, b):
    s = _compute_s(emb, W, b)
    out = _get_sc_kernel()(s, x.T)
    return out.reshape(BATCH, 1)
